# baseline (speedup 1.0000x reference)
import functools

import jax
import jax.numpy as jnp
from jax.experimental import pallas as pl
from jax.experimental.pallas import tpu as pltpu

_MiB = 1024 * 1024


def _se_kernel(x_ref, w1t_ref, w2t_ref, o_ref, *, inv_hw):
    x = x_ref[...]
    pooled = jnp.sum(x, axis=2, dtype=jnp.float32) * inv_hw
    hidden = jnp.maximum(
        jnp.dot(pooled, w1t_ref[...], preferred_element_type=jnp.float32), 0.0)
    s = jax.nn.sigmoid(
        jnp.dot(hidden, w2t_ref[...], preferred_element_type=jnp.float32))
    o_ref[...] = (x * s[:, :, None].astype(x.dtype)).astype(o_ref.dtype)


def _pick_batch_block(N, C, HW, itemsize):
    hw_pad = pl.cdiv(HW, 128) * 128
    for b in (16, 8, 4, 2, 1):
        if N % b == 0 and 4 * b * C * hw_pad * itemsize <= 40 * _MiB:
            return b
    return 1


def kernel(x, w1, w2):
    N, C, H, W = x.shape
    HW = H * W
    Cr = w1.shape[0]
    x_flat = x.reshape(N, C, HW)
    w1t = w1.astype(jnp.float32).T
    w2t = w2.astype(jnp.float32).T

    itemsize = jnp.dtype(x.dtype).itemsize
    B = _pick_batch_block(N, C, HW, itemsize)
    hw_pad = pl.cdiv(HW, 128) * 128
    block_bytes = 4 * B * C * hw_pad * itemsize
    w_bytes = 2 * C * Cr * 4
    cost = pl.CostEstimate(
        flops=int(2 * N * HW * C + 4 * N * C * Cr),
        transcendentals=int(N * C),
        bytes_accessed=int(2 * N * HW * C * itemsize + w_bytes),
    )
    out_flat = pl.pallas_call(
        functools.partial(_se_kernel, inv_hw=1.0 / HW),
        out_shape=jax.ShapeDtypeStruct((N, C, HW), x.dtype),
        grid=(N // B,),
        in_specs=[
            pl.BlockSpec((B, C, HW), lambda n: (n, 0, 0)),
            pl.BlockSpec((C, Cr), lambda n: (0, 0)),
            pl.BlockSpec((Cr, C), lambda n: (0, 0)),
        ],
        out_specs=pl.BlockSpec((B, C, HW), lambda n: (n, 0, 0)),
        compiler_params=pltpu.CompilerParams(
            dimension_semantics=("parallel",),
            vmem_limit_bytes=int(min(block_bytes + 4 * _MiB, 56 * _MiB)),
        ),
        cost_estimate=cost,
    )(x_flat, w1t, w2t)
    return out_flat.reshape(N, C, H, W)
